# baseline (speedup 1.0000x reference)
import functools

import jax
import jax.numpy as jnp
from jax import lax
from jax.experimental import pallas as pl
from jax.experimental.pallas import tpu as pltpu
from jax.experimental.pallas import tpu_sc as plsc

_ROWS_PER_CHUNK = 1


def _build_sc_call(num_out, in_features, num_cb, cb_size):
  info = plsc.get_sparse_core_info()
  nc, ns, L = info.num_cores, info.num_subcores, info.num_lanes
  nw = nc * ns
  rows_per_w = num_out // nw
  n_blocks = in_features // (L * 8)
  rpc = _ROWS_PER_CHUNK
  n_chunks = rows_per_w // rpc
  half_tbl = num_cb * cb_size
  mesh = plsc.VectorSubcoreMesh(core_axis_name="c", subcore_axis_name="s")

  @functools.partial(
      pl.kernel,
      mesh=mesh,
      out_type=jax.ShapeDtypeStruct((num_out * in_features,), jnp.float32),
      compiler_params=pltpu.CompilerParams(needs_layout_passes=False),
      scratch_types=[
          pltpu.VMEM((4 * half_tbl,), jnp.int32),
          pltpu.VMEM((rpc * in_features,), jnp.int32),
          pltpu.VMEM((rpc * in_features,), jnp.float32),
          pltpu.VMEM((rows_per_w,), jnp.float32),
      ],
  )
  def k(codes_hbm, tbl_hbm, scales_hbm, out_hbm, tbl_v, codes_v, out_v,
        scales_v):
    wid = lax.axis_index("s") * nc + lax.axis_index("c")
    row0 = wid * rows_per_w
    pltpu.sync_copy(tbl_hbm, tbl_v)
    pltpu.sync_copy(scales_hbm.at[pl.ds(row0, rows_per_w)], scales_v)
    lane8 = lax.iota(jnp.int32, L) * 8
    acc0 = jnp.zeros((2 * L,), jnp.bfloat16)

    def chunk_body(c, carry):
      cbase = (row0 + c * rpc) * in_features
      pltpu.sync_copy(codes_hbm.at[pl.ds(cbase, rpc * in_features)], codes_v)

      def row_body(rr, carry2):
        scale = plsc.load_gather(
            scales_v, [jnp.full((L,), c * rpc + rr, jnp.int32)])
        rbase = rr * in_features

        def blk_body(b, carry3):
          boff = rbase + b * (L * 8)
          code_idx = lane8 + boff
          accs = [jnp.zeros((L,), jnp.float32) for _ in range(8)]
          for m in range(num_cb):
            cm = plsc.load_gather(codes_v, [code_idx + m])
            for jp in range(4):
              g = plsc.load_gather(tbl_v, [cm + (jp * half_tbl + m * cb_size)])
              a, b2 = plsc.unpack(
                  plsc.bitcast(g, jnp.bfloat16),
                  format=plsc.PackFormat.INTERLEAVED)
              accs[2 * jp] = accs[2 * jp] + a
              accs[2 * jp + 1] = accs[2 * jp + 1] + b2
          for j in range(8):
            plsc.store_scatter(out_v, [code_idx + j], accs[j] * scale)
          return carry3

        lax.fori_loop(0, n_blocks, blk_body, 0, unroll=False)
        return carry2

      lax.fori_loop(0, rpc, row_body, 0, unroll=False)
      pltpu.sync_copy(out_v, out_hbm.at[pl.ds(cbase, rpc * in_features)])
      return carry

    lax.fori_loop(0, n_chunks, chunk_body, 0, unroll=False)

  return k


def kernel(codes, codebooks, scales):
  num_out, num_in_groups, num_cb = codes.shape
  _, cb_size, out_group, in_group = codebooks.shape
  in_features = num_in_groups * in_group

  codes_flat = codes.reshape(num_out * num_in_groups * num_cb)
  scales_flat = scales.reshape(num_out)
  cb = codebooks.reshape(num_cb, cb_size, in_group).astype(jnp.bfloat16)
  cb = cb.reshape(num_cb, cb_size, in_group // 2, 2).transpose(2, 0, 1, 3)
  tbl = lax.bitcast_convert_type(cb, jnp.int32).reshape(-1)

  call = _build_sc_call(num_out, in_features, num_cb, cb_size)
  out = call(codes_flat, tbl, scales_flat)
  return out.reshape(num_out, in_features)


# speedup vs baseline: 2.6000x; 2.6000x over previous
import functools

import jax
import jax.numpy as jnp
from jax import lax
from jax.experimental import pallas as pl
from jax.experimental.pallas import tpu as pltpu
from jax.experimental.pallas import tpu_sc as plsc


def _build_sc_call(num_out, in_features, num_cb, cb_size):
  info = plsc.get_sparse_core_info()
  nc, ns, L = info.num_cores, info.num_subcores, info.num_lanes
  nw = nc * ns
  rows_per_w = num_out // nw
  n_blocks = in_features // (L * 8)
  half_tbl = num_cb * cb_size
  mesh = plsc.VectorSubcoreMesh(core_axis_name="c", subcore_axis_name="s")

  @functools.partial(
      pl.kernel,
      mesh=mesh,
      out_type=jax.ShapeDtypeStruct((num_out, in_features), jnp.float32),
      compiler_params=pltpu.CompilerParams(needs_layout_passes=False),
      scratch_types=[
          pltpu.VMEM((4 * half_tbl,), jnp.int32),
          pltpu.VMEM((in_features,), jnp.int32),
          pltpu.VMEM((in_features,), jnp.float32),
          pltpu.VMEM((rows_per_w,), jnp.float32),
      ],
  )
  def k(codes_hbm, tbl_hbm, scales_hbm, out_hbm, tbl_v, codes_v, out_v,
        scales_v):
    wid = lax.axis_index("s") * nc + lax.axis_index("c")
    row0 = wid * rows_per_w
    pltpu.sync_copy(tbl_hbm, tbl_v)
    pltpu.sync_copy(scales_hbm.at[pl.ds(row0, rows_per_w)], scales_v)
    lane8 = lax.iota(jnp.int32, L) * 8

    def row_body(r, carry):
      pltpu.sync_copy(codes_hbm.at[row0 + r], codes_v)
      scale = plsc.load_gather(scales_v, [jnp.full((L,), r, jnp.int32)])

      def blk_body(b, carry2):
        code_idx = lane8 + b * (L * 8)
        accs = [jnp.zeros((L,), jnp.float32) for _ in range(8)]
        for m in range(num_cb):
          cm = plsc.load_gather(codes_v, [code_idx + m])
          for jp in range(4):
            g = plsc.load_gather(tbl_v, [cm + (jp * half_tbl + m * cb_size)])
            a, b2 = plsc.unpack(
                plsc.bitcast(g, jnp.bfloat16),
                format=plsc.PackFormat.INTERLEAVED)
            accs[2 * jp] = accs[2 * jp] + a
            accs[2 * jp + 1] = accs[2 * jp + 1] + b2
        for j in range(8):
          plsc.store_scatter(out_v, [code_idx + j], accs[j] * scale)
        return carry2

      lax.fori_loop(0, n_blocks, blk_body, 0, unroll=False)
      pltpu.sync_copy(out_v, out_hbm.at[row0 + r])
      return carry

    lax.fori_loop(0, rows_per_w, row_body, 0, unroll=False)

  return k


def kernel(codes, codebooks, scales):
  num_out, num_in_groups, num_cb = codes.shape
  _, cb_size, out_group, in_group = codebooks.shape
  in_features = num_in_groups * in_group

  codes_flat = codes.reshape(num_out, num_in_groups * num_cb)
  scales_flat = scales.reshape(num_out)
  cb = codebooks.reshape(num_cb, cb_size, in_group).astype(jnp.bfloat16)
  cb = cb.reshape(num_cb, cb_size, in_group // 2, 2).transpose(2, 0, 1, 3)
  tbl = lax.bitcast_convert_type(cb, jnp.int32).reshape(-1)

  call = _build_sc_call(num_out, in_features, num_cb, cb_size)
  return call(codes_flat, tbl, scales_flat)


# speedup vs baseline: 3.6229x; 1.3934x over previous
import functools

import jax
import jax.numpy as jnp
from jax import lax
from jax.experimental import pallas as pl
from jax.experimental.pallas import tpu as pltpu
from jax.experimental.pallas import tpu_sc as plsc


def _build_sc_call(num_out, in_features, num_cb, cb_size):
  info = plsc.get_sparse_core_info()
  nc, ns, L = info.num_cores, info.num_subcores, info.num_lanes
  nw = nc * ns
  rows_per_w = num_out // nw
  n_blocks = in_features // (L * 8)
  half_tbl = num_cb * cb_size
  mesh = plsc.VectorSubcoreMesh(core_axis_name="c", subcore_axis_name="s")

  @functools.partial(
      pl.kernel,
      mesh=mesh,
      out_type=jax.ShapeDtypeStruct((num_out, in_features), jnp.float32),
      compiler_params=pltpu.CompilerParams(needs_layout_passes=False),
      scratch_types=[
          pltpu.VMEM((4 * half_tbl,), jnp.int32),
          pltpu.VMEM((in_features,), jnp.int32),
          pltpu.VMEM((in_features,), jnp.int32),
          pltpu.VMEM((in_features,), jnp.float32),
          pltpu.VMEM((in_features,), jnp.float32),
          pltpu.VMEM((rows_per_w,), jnp.float32),
          pltpu.SemaphoreType.DMA,
          pltpu.SemaphoreType.DMA,
          pltpu.SemaphoreType.DMA,
          pltpu.SemaphoreType.DMA,
      ],
  )
  def k(codes_hbm, tbl_hbm, scales_hbm, out_hbm, tbl_v, codes_a, codes_b,
        out_a, out_b, scales_v, sem_ia, sem_ib, sem_oa, sem_ob):
    wid = lax.axis_index("s") * nc + lax.axis_index("c")
    row0 = wid * rows_per_w
    pltpu.sync_copy(tbl_hbm, tbl_v)
    pltpu.sync_copy(scales_hbm.at[pl.ds(row0, rows_per_w)], scales_v)
    lane8 = lax.iota(jnp.int32, L) * 8

    def compute_row(r, cv, ov):
      scale = plsc.load_gather(scales_v, [jnp.full((L,), r, jnp.int32)])

      def blk_body(b, carry2):
        code_idx = lane8 + b * (L * 8)
        accs = [jnp.zeros((L,), jnp.float32) for _ in range(8)]
        for m in range(num_cb):
          cm = plsc.load_gather(cv, [code_idx + m])
          for jp in range(4):
            g = plsc.load_gather(tbl_v, [cm + (jp * half_tbl + m * cb_size)])
            a, b2 = plsc.unpack(
                plsc.bitcast(g, jnp.bfloat16),
                format=plsc.PackFormat.INTERLEAVED)
            accs[2 * jp] = accs[2 * jp] + a
            accs[2 * jp + 1] = accs[2 * jp + 1] + b2
        for j in range(8):
          plsc.store_scatter(ov, [code_idx + j], accs[j] * scale)
        return carry2

      lax.fori_loop(0, n_blocks, blk_body, 0, unroll=False)

    def issue_in(r, cv, sem):
      pltpu.async_copy(codes_hbm.at[row0 + r], cv, sem)

    def wait_in(r, cv, sem):
      pltpu.make_async_copy(codes_hbm.at[row0 + r], cv, sem).wait()

    def issue_out(r, ov, sem):
      pltpu.async_copy(ov, out_hbm.at[row0 + r], sem)

    def wait_out(r, ov, sem):
      pltpu.make_async_copy(ov, out_hbm.at[row0 + r], sem).wait()

    issue_in(0, codes_a, sem_ia)
    issue_in(1, codes_b, sem_ib)
    wait_in(0, codes_a, sem_ia)
    compute_row(0, codes_a, out_a)
    issue_out(0, out_a, sem_oa)
    issue_in(2, codes_a, sem_ia)
    wait_in(1, codes_b, sem_ib)
    compute_row(1, codes_b, out_b)
    issue_out(1, out_b, sem_ob)
    issue_in(3, codes_b, sem_ib)

    def pair_body(p, carry):
      r0 = 2 * p
      wait_out(r0 - 2, out_a, sem_oa)
      wait_in(r0, codes_a, sem_ia)
      compute_row(r0, codes_a, out_a)
      issue_out(r0, out_a, sem_oa)
      issue_in(r0 + 2, codes_a, sem_ia)
      r1 = r0 + 1
      wait_out(r1 - 2, out_b, sem_ob)
      wait_in(r1, codes_b, sem_ib)
      compute_row(r1, codes_b, out_b)
      issue_out(r1, out_b, sem_ob)
      issue_in(r1 + 2, codes_b, sem_ib)
      return carry

    lax.fori_loop(1, rows_per_w // 2 - 1, pair_body, 0, unroll=False)

    lr = rows_per_w - 2
    wait_out(lr - 2, out_a, sem_oa)
    wait_in(lr, codes_a, sem_ia)
    compute_row(lr, codes_a, out_a)
    issue_out(lr, out_a, sem_oa)
    wait_out(lr - 1, out_b, sem_ob)
    wait_in(lr + 1, codes_b, sem_ib)
    compute_row(lr + 1, codes_b, out_b)
    issue_out(lr + 1, out_b, sem_ob)
    wait_out(lr, out_a, sem_oa)
    wait_out(lr + 1, out_b, sem_ob)

  return k


def kernel(codes, codebooks, scales):
  num_out, num_in_groups, num_cb = codes.shape
  _, cb_size, out_group, in_group = codebooks.shape
  in_features = num_in_groups * in_group

  codes_flat = codes.reshape(num_out, num_in_groups * num_cb)
  scales_flat = scales.reshape(num_out)
  cb = codebooks.reshape(num_cb, cb_size, in_group).astype(jnp.bfloat16)
  cb = cb.reshape(num_cb, cb_size, in_group // 2, 2).transpose(2, 0, 1, 3)
  tbl = lax.bitcast_convert_type(cb, jnp.int32).reshape(-1)

  call = _build_sc_call(num_out, in_features, num_cb, cb_size)
  return call(codes_flat, tbl, scales_flat)


# speedup vs baseline: 3.6378x; 1.0041x over previous
import functools

import jax
import jax.numpy as jnp
from jax import lax
from jax.experimental import pallas as pl
from jax.experimental.pallas import tpu as pltpu
from jax.experimental.pallas import tpu_sc as plsc


def _build_sc_call(num_out, in_features, num_cb, cb_size):
  info = plsc.get_sparse_core_info()
  nc, ns, L = info.num_cores, info.num_subcores, info.num_lanes
  nw = nc * ns
  rows_per_w = num_out // nw
  n_blocks = in_features // (L * 8)
  half_tbl = num_cb * cb_size
  mesh = plsc.VectorSubcoreMesh(core_axis_name="c", subcore_axis_name="s")

  @functools.partial(
      pl.kernel,
      mesh=mesh,
      out_type=jax.ShapeDtypeStruct((num_out, in_features), jnp.float32),
      compiler_params=pltpu.CompilerParams(needs_layout_passes=False),
      scratch_types=[
          pltpu.VMEM((4 * half_tbl,), jnp.int32),
          pltpu.VMEM((in_features,), jnp.int32),
          pltpu.VMEM((in_features,), jnp.int32),
          pltpu.VMEM((in_features,), jnp.float32),
          pltpu.VMEM((in_features,), jnp.float32),
          pltpu.VMEM((rows_per_w,), jnp.float32),
          pltpu.SemaphoreType.DMA,
          pltpu.SemaphoreType.DMA,
          pltpu.SemaphoreType.DMA,
          pltpu.SemaphoreType.DMA,
      ],
  )
  def k(codes_hbm, tbl_hbm, scales_hbm, out_hbm, tbl_v, codes_a, codes_b,
        out_a, out_b, scales_v, sem_ia, sem_ib, sem_oa, sem_ob):
    wid = lax.axis_index("s") * nc + lax.axis_index("c")
    row0 = wid * rows_per_w
    pltpu.sync_copy(tbl_hbm, tbl_v)
    pltpu.sync_copy(scales_hbm.at[pl.ds(row0, rows_per_w)], scales_v)
    lane8 = lax.iota(jnp.int32, L) * 8

    def compute_row(r, cv, ov):
      scale = plsc.load_gather(scales_v, [jnp.full((L,), r, jnp.int32)])

      def blk_body(b, carry2):
        code_idx = lane8 + b * (L * 8)
        accs = [jnp.zeros((L,), jnp.float32) for _ in range(8)]
        for m in range(num_cb):
          cm = plsc.load_gather(cv, [code_idx + m])
          for jp in range(4):
            g = plsc.load_gather(tbl_v, [cm + (jp * half_tbl + m * cb_size)])
            a, b2 = plsc.unpack(
                plsc.bitcast(g, jnp.bfloat16),
                format=plsc.PackFormat.INTERLEAVED)
            accs[2 * jp] = accs[2 * jp] + a
            accs[2 * jp + 1] = accs[2 * jp + 1] + b2
        for j in range(8):
          plsc.store_scatter(ov, [code_idx + j], accs[j] * scale)
        return carry2

      lax.fori_loop(0, n_blocks, blk_body, 0, unroll=False)

    def issue_in(r, cv, sem):
      pltpu.async_copy(codes_hbm.at[row0 + r], cv, sem)

    def wait_in(r, cv, sem):
      pltpu.make_async_copy(codes_hbm.at[row0 + r], cv, sem).wait()

    def issue_out(r, ov, sem):
      pltpu.async_copy(ov, out_hbm.at[row0 + r], sem)

    def wait_out(r, ov, sem):
      pltpu.make_async_copy(ov, out_hbm.at[row0 + r], sem).wait()

    n_pairs = rows_per_w // 2
    issue_in(0, codes_a, sem_ia)
    issue_in(1, codes_b, sem_ib)

    def pair_body(p, carry):
      r0 = 2 * p
      r1 = r0 + 1

      @pl.when(p > 0)
      def _():
        wait_out(r0 - 2, out_a, sem_oa)

      wait_in(r0, codes_a, sem_ia)
      compute_row(r0, codes_a, out_a)
      issue_out(r0, out_a, sem_oa)

      @pl.when(p < n_pairs - 1)
      def _():
        issue_in(r0 + 2, codes_a, sem_ia)

      @pl.when(p > 0)
      def _():
        wait_out(r1 - 2, out_b, sem_ob)

      wait_in(r1, codes_b, sem_ib)
      compute_row(r1, codes_b, out_b)
      issue_out(r1, out_b, sem_ob)

      @pl.when(p < n_pairs - 1)
      def _():
        issue_in(r1 + 2, codes_b, sem_ib)

      return carry

    lax.fori_loop(0, n_pairs, pair_body, 0, unroll=False)
    wait_out(rows_per_w - 2, out_a, sem_oa)
    wait_out(rows_per_w - 1, out_b, sem_ob)

  return k


def kernel(codes, codebooks, scales):
  num_out, num_in_groups, num_cb = codes.shape
  _, cb_size, out_group, in_group = codebooks.shape
  in_features = num_in_groups * in_group

  codes_flat = codes.reshape(num_out, num_in_groups * num_cb)
  scales_flat = scales.reshape(num_out)
  cb = codebooks.reshape(num_cb, cb_size, in_group).astype(jnp.bfloat16)
  cb = cb.reshape(num_cb, cb_size, in_group // 2, 2).transpose(2, 0, 1, 3)
  tbl = lax.bitcast_convert_type(cb, jnp.int32).reshape(-1)

  call = _build_sc_call(num_out, in_features, num_cb, cb_size)
  return call(codes_flat, tbl, scales_flat)


# speedup vs baseline: 3.9201x; 1.0776x over previous
import functools

import jax
import jax.numpy as jnp
from jax import lax
from jax.experimental import pallas as pl
from jax.experimental.pallas import tpu as pltpu
from jax.experimental.pallas import tpu_sc as plsc


def _build_sc_call(num_out, in_features, num_cb, cb_size):
  info = plsc.get_sparse_core_info()
  nc, ns, L = info.num_cores, info.num_subcores, info.num_lanes
  nw = nc * ns
  rows_per_w = num_out // nw
  n_blocks = in_features // (L * 8)
  half_tbl = num_cb * cb_size
  mesh = plsc.VectorSubcoreMesh(core_axis_name="c", subcore_axis_name="s")

  @functools.partial(
      pl.kernel,
      mesh=mesh,
      out_type=jax.ShapeDtypeStruct((num_out, in_features), jnp.float32),
      compiler_params=pltpu.CompilerParams(needs_layout_passes=False),
      scratch_types=[
          pltpu.VMEM((4 * half_tbl,), jnp.int32),
          pltpu.VMEM((in_features,), jnp.int32),
          pltpu.VMEM((in_features,), jnp.int32),
          pltpu.VMEM((in_features,), jnp.float32),
          pltpu.VMEM((in_features,), jnp.float32),
          pltpu.VMEM((rows_per_w,), jnp.float32),
          pltpu.SemaphoreType.DMA,
          pltpu.SemaphoreType.DMA,
          pltpu.SemaphoreType.DMA,
          pltpu.SemaphoreType.DMA,
      ],
  )
  def k(codes_hbm, tbl_hbm, scales_hbm, out_hbm, tbl_v, codes_a, codes_b,
        out_a, out_b, scales_v, sem_ia, sem_ib, sem_oa, sem_ob):
    wid = lax.axis_index("s") * nc + lax.axis_index("c")
    row0 = wid * rows_per_w
    pltpu.sync_copy(tbl_hbm, tbl_v)
    pltpu.sync_copy(scales_hbm.at[pl.ds(row0, rows_per_w)], scales_v)
    lane8 = lax.iota(jnp.int32, L) * 8

    def compute_row(r, cv, ov):
      scale = plsc.load_gather(scales_v, [jnp.full((L,), r, jnp.int32)])

      def blk_body(b, carry2):
        code_idx = lane8 + b * (L * 8)
        accs = [jnp.zeros((2 * L,), jnp.bfloat16) for _ in range(4)]
        for m in range(num_cb):
          cm = plsc.load_gather(cv, [code_idx + m])
          for jp in range(4):
            g = plsc.load_gather(tbl_v, [cm + (jp * half_tbl + m * cb_size)])
            accs[jp] = accs[jp] + plsc.bitcast(g, jnp.bfloat16)
        for jp in range(4):
          a, b2 = plsc.unpack(accs[jp], format=plsc.PackFormat.INTERLEAVED)
          plsc.store_scatter(ov, [code_idx + 2 * jp], a * scale)
          plsc.store_scatter(ov, [code_idx + (2 * jp + 1)], b2 * scale)
        return carry2

      lax.fori_loop(0, n_blocks, blk_body, 0, unroll=False)

    def issue_in(r, cv, sem):
      pltpu.async_copy(codes_hbm.at[row0 + r], cv, sem)

    def wait_in(r, cv, sem):
      pltpu.make_async_copy(codes_hbm.at[row0 + r], cv, sem).wait()

    def issue_out(r, ov, sem):
      pltpu.async_copy(ov, out_hbm.at[row0 + r], sem)

    def wait_out(r, ov, sem):
      pltpu.make_async_copy(ov, out_hbm.at[row0 + r], sem).wait()

    n_pairs = rows_per_w // 2
    issue_in(0, codes_a, sem_ia)
    issue_in(1, codes_b, sem_ib)

    def pair_body(p, carry):
      r0 = 2 * p
      r1 = r0 + 1

      @pl.when(p > 0)
      def _():
        wait_out(r0 - 2, out_a, sem_oa)

      wait_in(r0, codes_a, sem_ia)
      compute_row(r0, codes_a, out_a)
      issue_out(r0, out_a, sem_oa)

      @pl.when(p < n_pairs - 1)
      def _():
        issue_in(r0 + 2, codes_a, sem_ia)

      @pl.when(p > 0)
      def _():
        wait_out(r1 - 2, out_b, sem_ob)

      wait_in(r1, codes_b, sem_ib)
      compute_row(r1, codes_b, out_b)
      issue_out(r1, out_b, sem_ob)

      @pl.when(p < n_pairs - 1)
      def _():
        issue_in(r1 + 2, codes_b, sem_ib)

      return carry

    lax.fori_loop(0, n_pairs, pair_body, 0, unroll=False)
    wait_out(rows_per_w - 2, out_a, sem_oa)
    wait_out(rows_per_w - 1, out_b, sem_ob)

  return k


def kernel(codes, codebooks, scales):
  num_out, num_in_groups, num_cb = codes.shape
  _, cb_size, out_group, in_group = codebooks.shape
  in_features = num_in_groups * in_group

  codes_flat = codes.reshape(num_out, num_in_groups * num_cb)
  scales_flat = scales.reshape(num_out)
  cb = codebooks.reshape(num_cb, cb_size, in_group).astype(jnp.bfloat16)
  cb = cb.reshape(num_cb, cb_size, in_group // 2, 2).transpose(2, 0, 1, 3)
  tbl = lax.bitcast_convert_type(cb, jnp.int32).reshape(-1)

  call = _build_sc_call(num_out, in_features, num_cb, cb_size)
  return call(codes_flat, tbl, scales_flat)
